# baseline (speedup 1.0000x reference)
import functools

import jax
import jax.numpy as jnp
from jax import lax
from jax.experimental import pallas as pl
from jax.experimental.pallas import tpu as pltpu
from jax.experimental.pallas import tpu_sc as plsc

N = 100000
NPAD = 100352
E = 3200000
EPAD = 3211264
RWS = EPAD // 256
NC, NS = 2, 16
NR1 = RWS // NS
NR2 = RWS // (NC * NS)
CH = 8
NCH1 = NR1 // CH
NCH2 = NR2 // CH
DR = NPAD // NS

_mesh = plsc.VectorSubcoreMesh(core_axis_name="c", subcore_axis_name="s")
_f32 = jnp.float32
_SC_PARAMS = pltpu.CompilerParams(use_tc_tiling_on_sc=False)

_NB = 4
_LEAD = 2


def _process_chunk(table, sv, dv, rows, gsem, ssem, acc):
    for j in range(_LEAD):
        pltpu.async_copy(table.at[sv.at[j]], rows.at[j], gsem[j])

    @pl.loop(0, CH // _NB)
    def _(q):
        k = q * _NB
        for j in range(_NB):
            i = k + j
            s2 = (j + _LEAD) % _NB
            pltpu.make_async_copy(table.at[sv.at[i]], rows.at[j],
                                  gsem[j]).wait()
            pltpu.async_copy(rows.at[j], acc.at[dv.at[i]], ssem[j],
                             add=True)

            @pl.when(i + _LEAD < CH)
            def _():
                @pl.when(i + _LEAD >= _NB)
                def _():
                    pltpu.make_async_copy(rows.at[s2], acc.at[dv.at[i]],
                                          ssem[s2]).wait()

                pltpu.async_copy(table.at[sv.at[i + _LEAD]], rows.at[s2],
                                 gsem[s2])

    for j in range(_NB):
        pltpu.make_async_copy(rows.at[j], acc.at[dv.at[0]], ssem[j]).wait()


def _edge_loop(table, src_hbm, dst_hbm, r0, nch, sv, dv, rows, gsem, ssem,
               acc):
    @pl.loop(0, nch)
    def _(c):
        base = r0 + c * CH
        pltpu.sync_copy(src_hbm.at[pl.ds(base, CH)], sv)
        pltpu.sync_copy(dst_hbm.at[pl.ds(base, CH)], dv)
        _process_chunk(table, sv, dv, rows, gsem, ssem, acc)


@functools.partial(
    pl.kernel,
    out_type=jax.ShapeDtypeStruct((NC * NPAD,), _f32),
    mesh=_mesh,
    scratch_types=[
        pltpu.VMEM((CH, 256), jnp.int32),
        pltpu.VMEM((256,), _f32),
        pltpu.SemaphoreType.DMA,
        pltpu.VMEM_SHARED((NPAD,), _f32),
    ],
    compiler_params=_SC_PARAMS,
)
def _deg_kernel(dst_hbm, zeros_hbm, out_hbm, dst_v, ones_v, dsem, acc):
    cid = lax.axis_index("c")
    sid = lax.axis_index("s")
    for i in range(16):
        ones_v[pl.ds(i * 16, 16)] = jnp.full((16,), 1.0, _f32)
    pltpu.sync_copy(zeros_hbm.at[pl.ds(sid * DR, DR)],
                    acc.at[pl.ds(sid * DR, DR)])
    plsc.subcore_barrier()

    r0 = (cid * NS + sid) * NR2

    @pl.loop(0, NCH2)
    def _(c):
        pltpu.sync_copy(dst_hbm.at[pl.ds(r0 + c * CH, CH)], dst_v)

        @pl.loop(0, 8)
        def _(k):
            pltpu.async_copy(ones_v, acc.at[dst_v.at[k]], dsem, add=True)

        @pl.loop(0, CH - 8)
        def _(k):
            pltpu.async_copy(ones_v, acc.at[dst_v.at[k + 8]], dsem,
                             add=True)
            pltpu.make_async_copy(ones_v, acc.at[dst_v.at[0]], dsem).wait()

        @pl.loop(0, 8)
        def _(k):
            pltpu.make_async_copy(ones_v, acc.at[dst_v.at[0]], dsem).wait()

    plsc.subcore_barrier()
    pltpu.sync_copy(acc.at[pl.ds(sid * DR, DR)],
                    out_hbm.at[pl.ds(cid * NPAD + sid * DR, DR)])


_AGG_SCRATCH = (
    [pltpu.VMEM((CH, 256), jnp.int32),
     pltpu.VMEM((CH, 256), jnp.int32),
     pltpu.VMEM((_NB, 256, 16), _f32)]
    + [pltpu.SemaphoreType.DMA] * (2 * _NB)
    + [pltpu.VMEM_SHARED((NPAD, 16), _f32)]
)


@functools.partial(
    pl.kernel,
    out_type=jax.ShapeDtypeStruct((NC * NPAD, 16), _f32),
    mesh=_mesh,
    scratch_types=_AGG_SCRATCH,
    compiler_params=_SC_PARAMS,
)
def _agg1_kernel(src_hbm, dst_hbm, z0_hbm, z1_hbm, zeros_hbm, out_hbm,
                 sv, dv, rows, *sems_acc):
    gsem = list(sems_acc[:_NB])
    ssem = list(sems_acc[_NB:2 * _NB])
    acc = sems_acc[2 * _NB]
    cid = lax.axis_index("c")
    sid = lax.axis_index("s")
    pltpu.sync_copy(zeros_hbm.at[pl.ds(sid * DR, DR)],
                    acc.at[pl.ds(sid * DR, DR)])
    plsc.subcore_barrier()

    r0 = sid * NR1

    @pl.when(cid == 0)
    def _():
        _edge_loop(z0_hbm, src_hbm, dst_hbm, r0, NCH1, sv, dv, rows,
                   gsem, ssem, acc)

    @pl.when(cid == 1)
    def _():
        _edge_loop(z1_hbm, src_hbm, dst_hbm, r0, NCH1, sv, dv, rows,
                   gsem, ssem, acc)

    plsc.subcore_barrier()
    pltpu.sync_copy(acc.at[pl.ds(sid * DR, DR)],
                    out_hbm.at[pl.ds(cid * NPAD + sid * DR, DR)])


@functools.partial(
    pl.kernel,
    out_type=jax.ShapeDtypeStruct((NC * NPAD, 16), _f32),
    mesh=_mesh,
    scratch_types=_AGG_SCRATCH,
    compiler_params=_SC_PARAMS,
)
def _agg2_kernel(src_hbm, dst_hbm, z_hbm, zeros_hbm, out_hbm,
                 sv, dv, rows, *sems_acc):
    gsem = list(sems_acc[:_NB])
    ssem = list(sems_acc[_NB:2 * _NB])
    acc = sems_acc[2 * _NB]
    cid = lax.axis_index("c")
    sid = lax.axis_index("s")
    pltpu.sync_copy(zeros_hbm.at[pl.ds(sid * DR, DR)],
                    acc.at[pl.ds(sid * DR, DR)])
    plsc.subcore_barrier()

    r0 = (cid * NS + sid) * NR2
    _edge_loop(z_hbm, src_hbm, dst_hbm, r0, NCH2, sv, dv, rows,
               gsem, ssem, acc)

    plsc.subcore_barrier()
    pltpu.sync_copy(acc.at[pl.ds(sid * DR, DR)],
                    out_hbm.at[pl.ds(cid * NPAD + sid * DR, DR)])


_R = 1024
_G = NPAD // _R


def _mm_body(x_ref, w_ref, o_ref):
    o_ref[...] = jnp.dot(x_ref[...], w_ref[...],
                         preferred_element_type=_f32)


def _srow(d0_ref, d1_ref):
    deg = d0_ref[...] + d1_ref[...] + 1.0
    return lax.rsqrt(deg)[:, None]


def _z1_body(d0_ref, d1_ref, xw_ref, z0_ref, z1_ref):
    z = xw_ref[...] * _srow(d0_ref, d1_ref)
    z0_ref[...] = z[:, :16]
    z1_ref[...] = z[:, 16:]


def _z2_body(d0_ref, d1_ref, agg0_ref, agg1_ref, z0_ref, z1_ref, b1_ref,
             w2_ref, o_ref):
    agg = jnp.concatenate([agg0_ref[...], agg1_ref[...]], axis=1)
    z = jnp.concatenate([z0_ref[...], z1_ref[...]], axis=1)
    s2 = _srow(d0_ref, d1_ref)
    h = jnp.maximum(s2 * (agg + z) + b1_ref[...], 0.0)
    o_ref[...] = s2 * jnp.dot(h, w2_ref[...], preferred_element_type=_f32)


def _out_body(d0_ref, d1_ref, agg0_ref, agg1_ref, z_ref, b2_ref, wfc_ref,
              bfc_ref, o_ref):
    s2 = _srow(d0_ref, d1_ref)
    h = jnp.maximum(
        s2 * (agg0_ref[...] + agg1_ref[...] + z_ref[...]) + b2_ref[...], 0.0)
    o_ref[...] = (jnp.sum(h * wfc_ref[...], axis=1, keepdims=True)
                  + bfc_ref[0, 0])


def _row_spec(w):
    return pl.BlockSpec((_R, w), lambda i: (i, 0))


def _full_spec(shape):
    return pl.BlockSpec(shape, lambda i: tuple(0 for _ in shape))


def _half_spec(w, half):
    return pl.BlockSpec((_R, w), lambda i, h=half: (h * _G + i, 0))


def _deg_spec(half):
    return pl.BlockSpec((_R,), lambda i, h=half: (h * _G + i,))


def kernel(edge_index, node_features, W1, b1, W2, b2, Wfc, bfc):
    src = edge_index[0]
    dst = edge_index[1]
    pad = N + (jnp.arange(EPAD - E, dtype=src.dtype) % 64)
    src_r = jnp.concatenate([src, pad]).reshape(RWS, 256)
    dst_r = jnp.concatenate([dst, pad]).reshape(RWS, 256)
    zeros1 = jnp.zeros((NPAD,), _f32)
    zeros2 = jnp.zeros((NPAD, 16), _f32)

    xw1 = pl.pallas_call(
        _mm_body,
        grid=(_G,),
        in_specs=[_row_spec(16), _full_spec((16, 32))],
        out_specs=_row_spec(32),
        out_shape=jax.ShapeDtypeStruct((NPAD, 32), _f32),
    )(node_features, W1)

    deg2 = _deg_kernel(dst_r, zeros1)

    z0, z1 = pl.pallas_call(
        _z1_body,
        grid=(_G,),
        in_specs=[_deg_spec(0), _deg_spec(1), _row_spec(32)],
        out_specs=[_row_spec(16), _row_spec(16)],
        out_shape=[jax.ShapeDtypeStruct((NPAD, 16), _f32)] * 2,
    )(deg2, deg2, xw1)

    agg1 = _agg1_kernel(src_r, dst_r, z0, z1, zeros2)

    z2 = pl.pallas_call(
        _z2_body,
        grid=(_G,),
        in_specs=[_deg_spec(0), _deg_spec(1), _half_spec(16, 0),
                  _half_spec(16, 1), _row_spec(16), _row_spec(16),
                  _full_spec((1, 32)), _full_spec((32, 16))],
        out_specs=_row_spec(16),
        out_shape=jax.ShapeDtypeStruct((NPAD, 16), _f32),
    )(deg2, deg2, agg1, agg1, z0, z1, b1.reshape(1, 32), W2)

    agg2 = _agg2_kernel(src_r, dst_r, z2, zeros2)

    out = pl.pallas_call(
        _out_body,
        grid=(_G,),
        in_specs=[_deg_spec(0), _deg_spec(1), _half_spec(16, 0),
                  _half_spec(16, 1), _row_spec(16), _full_spec((1, 16)),
                  _full_spec((1, 16)), _full_spec((1, 1))],
        out_specs=_row_spec(1),
        out_shape=jax.ShapeDtypeStruct((NPAD, 1), _f32),
    )(deg2, deg2, agg2, agg2, z2, b2.reshape(1, 16), Wfc.reshape(1, 16),
      bfc.reshape(1, 1))

    return out[:N]


# speedup vs baseline: 1.9090x; 1.9090x over previous
import functools

import jax
import jax.numpy as jnp
from jax import lax
from jax.experimental import pallas as pl
from jax.experimental.pallas import tpu as pltpu
from jax.experimental.pallas import tpu_sc as plsc

N = 100000
NPAD = 100352
E = 3200000
EPAD = 3211264
RWS = EPAD // 128
NC, NS = 2, 16
NR1 = RWS // NS
NR2 = RWS // (NC * NS)
CH = 56
NCH1 = NR1 // CH
NCH2 = NR2 // CH
DR = NPAD // NS

_mesh = plsc.VectorSubcoreMesh(core_axis_name="c", subcore_axis_name="s")
_f32 = jnp.float32
_SC_PARAMS = pltpu.CompilerParams(use_tc_tiling_on_sc=False)

_NB = 7
_LEAD = 4


def _process_chunk(table, sv, dv, rows, gsem, ssem, acc):
    for j in range(_LEAD):
        pltpu.async_copy(table.at[sv.at[j]], rows.at[j], gsem[j])

    @pl.loop(0, CH // _NB)
    def _(q):
        k = q * _NB
        for j in range(_NB):
            i = k + j
            s2 = (j + _LEAD) % _NB
            pltpu.make_async_copy(table.at[sv.at[i]], rows.at[j],
                                  gsem[j]).wait()
            pltpu.async_copy(rows.at[j], acc.at[dv.at[i]], ssem[j],
                             add=True)

            @pl.when(i + _LEAD < CH)
            def _():
                @pl.when(i + _LEAD >= _NB)
                def _():
                    pltpu.make_async_copy(rows.at[s2], acc.at[dv.at[i]],
                                          ssem[s2]).wait()

                pltpu.async_copy(table.at[sv.at[i + _LEAD]], rows.at[s2],
                                 gsem[s2])

    for j in range(_NB):
        pltpu.make_async_copy(rows.at[j], acc.at[dv.at[0]], ssem[j]).wait()


def _edge_loop(table, src_hbm, dst_hbm, r0, nch, sv, dv, rows, gsem, ssem,
               acc):
    @pl.loop(0, nch)
    def _(c):
        base = r0 + c * CH
        pltpu.sync_copy(src_hbm.at[pl.ds(base, CH)], sv)
        pltpu.sync_copy(dst_hbm.at[pl.ds(base, CH)], dv)
        _process_chunk(table, sv, dv, rows, gsem, ssem, acc)


@functools.partial(
    pl.kernel,
    out_type=jax.ShapeDtypeStruct((NC * NPAD,), _f32),
    mesh=_mesh,
    scratch_types=[
        pltpu.VMEM((CH, 128), jnp.int32),
        pltpu.VMEM((128,), _f32),
        pltpu.SemaphoreType.DMA,
        pltpu.VMEM_SHARED((NPAD,), _f32),
    ],
    compiler_params=_SC_PARAMS,
)
def _deg_kernel(dst_hbm, zeros_hbm, out_hbm, dst_v, ones_v, dsem, acc):
    cid = lax.axis_index("c")
    sid = lax.axis_index("s")
    for i in range(8):
        ones_v[pl.ds(i * 16, 16)] = jnp.full((16,), 1.0, _f32)
    pltpu.sync_copy(zeros_hbm.at[pl.ds(sid * DR, DR)],
                    acc.at[pl.ds(sid * DR, DR)])
    plsc.subcore_barrier()

    r0 = (cid * NS + sid) * NR2

    @pl.loop(0, NCH2)
    def _(c):
        pltpu.sync_copy(dst_hbm.at[pl.ds(r0 + c * CH, CH)], dst_v)

        @pl.loop(0, 8)
        def _(k):
            pltpu.async_copy(ones_v, acc.at[dst_v.at[k]], dsem, add=True)

        @pl.loop(0, CH - 8)
        def _(k):
            pltpu.async_copy(ones_v, acc.at[dst_v.at[k + 8]], dsem,
                             add=True)
            pltpu.make_async_copy(ones_v, acc.at[dst_v.at[0]], dsem).wait()

        @pl.loop(0, 8)
        def _(k):
            pltpu.make_async_copy(ones_v, acc.at[dst_v.at[0]], dsem).wait()

    plsc.subcore_barrier()
    pltpu.sync_copy(acc.at[pl.ds(sid * DR, DR)],
                    out_hbm.at[pl.ds(cid * NPAD + sid * DR, DR)])


_AGG_SCRATCH = (
    [pltpu.VMEM((CH, 128), jnp.int32),
     pltpu.VMEM((CH, 128), jnp.int32),
     pltpu.VMEM((_NB, 128, 16), _f32)]
    + [pltpu.SemaphoreType.DMA] * (2 * _NB)
    + [pltpu.VMEM_SHARED((NPAD, 16), _f32)]
)


@functools.partial(
    pl.kernel,
    out_type=jax.ShapeDtypeStruct((NC * NPAD, 16), _f32),
    mesh=_mesh,
    scratch_types=_AGG_SCRATCH,
    compiler_params=_SC_PARAMS,
)
def _agg1_kernel(src_hbm, dst_hbm, z0_hbm, z1_hbm, zeros_hbm, out_hbm,
                 sv, dv, rows, *sems_acc):
    gsem = list(sems_acc[:_NB])
    ssem = list(sems_acc[_NB:2 * _NB])
    acc = sems_acc[2 * _NB]
    cid = lax.axis_index("c")
    sid = lax.axis_index("s")
    pltpu.sync_copy(zeros_hbm.at[pl.ds(sid * DR, DR)],
                    acc.at[pl.ds(sid * DR, DR)])
    plsc.subcore_barrier()

    r0 = sid * NR1

    @pl.when(cid == 0)
    def _():
        _edge_loop(z0_hbm, src_hbm, dst_hbm, r0, NCH1, sv, dv, rows,
                   gsem, ssem, acc)

    @pl.when(cid == 1)
    def _():
        _edge_loop(z1_hbm, src_hbm, dst_hbm, r0, NCH1, sv, dv, rows,
                   gsem, ssem, acc)

    plsc.subcore_barrier()
    pltpu.sync_copy(acc.at[pl.ds(sid * DR, DR)],
                    out_hbm.at[pl.ds(cid * NPAD + sid * DR, DR)])


@functools.partial(
    pl.kernel,
    out_type=jax.ShapeDtypeStruct((NC * NPAD, 16), _f32),
    mesh=_mesh,
    scratch_types=_AGG_SCRATCH,
    compiler_params=_SC_PARAMS,
)
def _agg2_kernel(src_hbm, dst_hbm, z_hbm, zeros_hbm, out_hbm,
                 sv, dv, rows, *sems_acc):
    gsem = list(sems_acc[:_NB])
    ssem = list(sems_acc[_NB:2 * _NB])
    acc = sems_acc[2 * _NB]
    cid = lax.axis_index("c")
    sid = lax.axis_index("s")
    pltpu.sync_copy(zeros_hbm.at[pl.ds(sid * DR, DR)],
                    acc.at[pl.ds(sid * DR, DR)])
    plsc.subcore_barrier()

    r0 = (cid * NS + sid) * NR2
    _edge_loop(z_hbm, src_hbm, dst_hbm, r0, NCH2, sv, dv, rows,
               gsem, ssem, acc)

    plsc.subcore_barrier()
    pltpu.sync_copy(acc.at[pl.ds(sid * DR, DR)],
                    out_hbm.at[pl.ds(cid * NPAD + sid * DR, DR)])


WN = NPAD * 16 // 128
SROWS = NPAD // 128
_RW = 1568
_GW = WN // _RW


def _mm_body(x_ref, wa_ref, wb_ref, oa_ref, ob_ref):
    x = x_ref[...]
    oa_ref[...] = jnp.dot(x, wa_ref[...], preferred_element_type=_f32)
    ob_ref[...] = jnp.dot(x, wb_ref[...], preferred_element_type=_f32)


def _s_body(deg_ref, s_ref):
    s_ref[...] = lax.rsqrt(deg_ref[0] + deg_ref[1] + 1.0)


def _zscale_body(xw0_ref, xw1_ref, s_ref, z0_ref, z1_ref):
    s = s_ref[...]
    z0_ref[...] = xw0_ref[...] * s
    z1_ref[...] = xw1_ref[...] * s


def _z2_body(a0_ref, a1_ref, z0_ref, z1_ref, s_ref, b1lo_ref, b1hi_ref,
             w2a_ref, w2b_ref, o_ref):
    s = s_ref[...]
    hlo = jnp.maximum(s * (a0_ref[...] + z0_ref[...]) + b1lo_ref[...], 0.0)
    hhi = jnp.maximum(s * (a1_ref[...] + z1_ref[...]) + b1hi_ref[...], 0.0)
    o_ref[...] = s * (jnp.dot(hlo, w2a_ref[...], preferred_element_type=_f32)
                      + jnp.dot(hhi, w2b_ref[...],
                                preferred_element_type=_f32))


def _out_body(a0_ref, a1_ref, z_ref, s_ref, b2_ref, wf_ref, bfc_ref, o_ref):
    h = jnp.maximum(
        s_ref[...] * (a0_ref[...] + a1_ref[...] + z_ref[...]) + b2_ref[...],
        0.0)
    o_ref[...] = (jnp.dot(h, wf_ref[...], preferred_element_type=_f32)
                  + bfc_ref[0, 0])


def _wspec(w=128):
    return pl.BlockSpec((_RW, w), lambda i: (i, 0))


def _whalf(half):
    return pl.BlockSpec((_RW, 128), lambda i, h=half: (h * _GW + i, 0))


def _full_spec(shape):
    return pl.BlockSpec(shape, lambda i: tuple(0 for _ in shape))


def kernel(edge_index, node_features, W1, b1, W2, b2, Wfc, bfc):
    src = edge_index[0]
    dst = edge_index[1]
    pad = N + (jnp.arange(EPAD - E, dtype=src.dtype) % 64)
    src_r = jnp.concatenate([src, pad]).reshape(RWS, 128)
    dst_r = jnp.concatenate([dst, pad]).reshape(RWS, 128)
    zeros1 = jnp.zeros((NPAD,), _f32)
    zeros2 = jnp.zeros((NPAD, 16), _f32)

    x_w = jnp.pad(node_features, ((0, NPAD - N), (0, 0))).reshape(WN, 128)
    e8 = jnp.eye(8, dtype=_f32)
    w1a = jnp.kron(e8, W1[:, :16])
    w1b = jnp.kron(e8, W1[:, 16:])
    w2a = jnp.kron(e8, W2[:16, :])
    w2b = jnp.kron(e8, W2[16:, :])
    wfk = jnp.kron(e8, Wfc)
    b1lo = jnp.tile(b1[:16], 8).reshape(1, 128)
    b1hi = jnp.tile(b1[16:], 8).reshape(1, 128)
    b2t = jnp.tile(b2, 8).reshape(1, 128)

    xw0, xw1 = pl.pallas_call(
        _mm_body,
        grid=(_GW,),
        in_specs=[_wspec(), _full_spec((128, 128)), _full_spec((128, 128))],
        out_specs=[_wspec(), _wspec()],
        out_shape=[jax.ShapeDtypeStruct((WN, 128), _f32)] * 2,
    )(x_w, w1a, w1b)

    deg2 = _deg_kernel(dst_r, zeros1).reshape(2, SROWS, 128)

    s_w = pl.pallas_call(
        _s_body,
        grid=(1,),
        in_specs=[_full_spec((2, SROWS, 128))],
        out_specs=_full_spec((SROWS, 128)),
        out_shape=jax.ShapeDtypeStruct((SROWS, 128), _f32),
    )(deg2)
    s16 = jnp.repeat(s_w.reshape(-1), 16).reshape(WN, 128)

    z0, z1 = pl.pallas_call(
        _zscale_body,
        grid=(_GW,),
        in_specs=[_wspec(), _wspec(), _wspec()],
        out_specs=[_wspec(), _wspec()],
        out_shape=[jax.ShapeDtypeStruct((WN, 128), _f32)] * 2,
    )(xw0, xw1, s16)

    agg1 = _agg1_kernel(src_r, dst_r, z0.reshape(NPAD, 16),
                        z1.reshape(NPAD, 16), zeros2).reshape(2 * WN, 128)

    z2 = pl.pallas_call(
        _z2_body,
        grid=(_GW,),
        in_specs=[_whalf(0), _whalf(1), _wspec(), _wspec(), _wspec(),
                  _full_spec((1, 128)), _full_spec((1, 128)),
                  _full_spec((128, 128)), _full_spec((128, 128))],
        out_specs=_wspec(),
        out_shape=jax.ShapeDtypeStruct((WN, 128), _f32),
    )(agg1, agg1, z0, z1, s16, b1lo, b1hi, w2a, w2b)

    agg2 = _agg2_kernel(src_r, dst_r, z2.reshape(NPAD, 16),
                        zeros2).reshape(2 * WN, 128)

    out = pl.pallas_call(
        _out_body,
        grid=(_GW,),
        in_specs=[_whalf(0), _whalf(1), _wspec(), _wspec(),
                  _full_spec((1, 128)), _full_spec((128, 8)),
                  _full_spec((1, 1))],
        out_specs=_wspec(8),
        out_shape=jax.ShapeDtypeStruct((WN, 8), _f32),
    )(agg2, agg2, z2, s16, b2t, wfk, bfc.reshape(1, 1))

    return out.reshape(NPAD, 1)[:N]


# speedup vs baseline: 2.0991x; 1.0996x over previous
import functools

import jax
import jax.numpy as jnp
from jax import lax
from jax.experimental import pallas as pl
from jax.experimental.pallas import tpu as pltpu
from jax.experimental.pallas import tpu_sc as plsc

N = 100000
NPAD = 100352
E = 3200000
EPAD = 3211264
RWS = EPAD // 128
NC, NS = 2, 16
NR1 = RWS // NS
NR2 = RWS // (NC * NS)
CH = 56
NCH1 = NR1 // CH
NCH2 = NR2 // CH
DR = NPAD // NS

_mesh = plsc.VectorSubcoreMesh(core_axis_name="c", subcore_axis_name="s")
_f32 = jnp.float32
_SC_PARAMS = pltpu.CompilerParams(use_tc_tiling_on_sc=False)

_NB = 7
_LEAD = 5


def _process_chunk(table, sv, dv, rows, gsem, ssem, acc):
    for j in range(_LEAD):
        pltpu.async_copy(table.at[sv.at[j]], rows.at[j], gsem[j])

    @pl.loop(0, CH // _NB)
    def _(q):
        k = q * _NB
        for j in range(_NB):
            i = k + j
            s2 = (j + _LEAD) % _NB
            pltpu.make_async_copy(table.at[sv.at[i]], rows.at[j],
                                  gsem[j]).wait()
            pltpu.async_copy(rows.at[j], acc.at[dv.at[i]], ssem[j],
                             add=True)

            @pl.when(i + _LEAD < CH)
            def _():
                @pl.when(i + _LEAD >= _NB)
                def _():
                    pltpu.make_async_copy(rows.at[s2], acc.at[dv.at[i]],
                                          ssem[s2]).wait()

                pltpu.async_copy(table.at[sv.at[i + _LEAD]], rows.at[s2],
                                 gsem[s2])

    for j in range(_NB):
        pltpu.make_async_copy(rows.at[j], acc.at[dv.at[0]], ssem[j]).wait()


def _edge_loop(table, src_hbm, dst_hbm, r0, nch, sv, dv, rows, gsem, ssem,
               acc):
    @pl.loop(0, nch)
    def _(c):
        base = r0 + c * CH
        pltpu.sync_copy(src_hbm.at[pl.ds(base, CH)], sv)
        pltpu.sync_copy(dst_hbm.at[pl.ds(base, CH)], dv)
        _process_chunk(table, sv, dv, rows, gsem, ssem, acc)


@functools.partial(
    pl.kernel,
    out_type=jax.ShapeDtypeStruct((NC * NPAD,), _f32),
    mesh=_mesh,
    scratch_types=[
        pltpu.VMEM((CH, 128), jnp.int32),
        pltpu.VMEM((128,), _f32),
        pltpu.SemaphoreType.DMA,
        pltpu.VMEM_SHARED((NPAD,), _f32),
    ],
    compiler_params=_SC_PARAMS,
)
def _deg_kernel(dst_hbm, zeros_hbm, out_hbm, dst_v, ones_v, dsem, acc):
    cid = lax.axis_index("c")
    sid = lax.axis_index("s")
    for i in range(8):
        ones_v[pl.ds(i * 16, 16)] = jnp.full((16,), 1.0, _f32)
    pltpu.sync_copy(zeros_hbm.at[pl.ds(sid * DR, DR)],
                    acc.at[pl.ds(sid * DR, DR)])
    plsc.subcore_barrier()

    r0 = (cid * NS + sid) * NR2

    @pl.loop(0, NCH2)
    def _(c):
        pltpu.sync_copy(dst_hbm.at[pl.ds(r0 + c * CH, CH)], dst_v)

        @pl.loop(0, 16)
        def _(k):
            pltpu.async_copy(ones_v, acc.at[dst_v.at[k]], dsem, add=True)

        @pl.loop(0, CH - 16)
        def _(k):
            pltpu.async_copy(ones_v, acc.at[dst_v.at[k + 16]], dsem,
                             add=True)
            pltpu.make_async_copy(ones_v, acc.at[dst_v.at[0]], dsem).wait()

        @pl.loop(0, 16)
        def _(k):
            pltpu.make_async_copy(ones_v, acc.at[dst_v.at[0]], dsem).wait()

    plsc.subcore_barrier()
    pltpu.sync_copy(acc.at[pl.ds(sid * DR, DR)],
                    out_hbm.at[pl.ds(cid * NPAD + sid * DR, DR)])


_AGG_SCRATCH = (
    [pltpu.VMEM((CH, 128), jnp.int32),
     pltpu.VMEM((CH, 128), jnp.int32),
     pltpu.VMEM((_NB, 128, 16), _f32)]
    + [pltpu.SemaphoreType.DMA] * (2 * _NB)
    + [pltpu.VMEM_SHARED((NPAD, 16), _f32)]
)


@functools.partial(
    pl.kernel,
    out_type=jax.ShapeDtypeStruct((NC * NPAD, 16), _f32),
    mesh=_mesh,
    scratch_types=_AGG_SCRATCH,
    compiler_params=_SC_PARAMS,
)
def _agg1_kernel(src_hbm, dst_hbm, z0_hbm, z1_hbm, zeros_hbm, out_hbm,
                 sv, dv, rows, *sems_acc):
    gsem = list(sems_acc[:_NB])
    ssem = list(sems_acc[_NB:2 * _NB])
    acc = sems_acc[2 * _NB]
    cid = lax.axis_index("c")
    sid = lax.axis_index("s")
    pltpu.sync_copy(zeros_hbm.at[pl.ds(sid * DR, DR)],
                    acc.at[pl.ds(sid * DR, DR)])
    plsc.subcore_barrier()

    r0 = sid * NR1

    @pl.when(cid == 0)
    def _():
        _edge_loop(z0_hbm, src_hbm, dst_hbm, r0, NCH1, sv, dv, rows,
                   gsem, ssem, acc)

    @pl.when(cid == 1)
    def _():
        _edge_loop(z1_hbm, src_hbm, dst_hbm, r0, NCH1, sv, dv, rows,
                   gsem, ssem, acc)

    plsc.subcore_barrier()
    pltpu.sync_copy(acc.at[pl.ds(sid * DR, DR)],
                    out_hbm.at[pl.ds(cid * NPAD + sid * DR, DR)])


@functools.partial(
    pl.kernel,
    out_type=jax.ShapeDtypeStruct((NC * NPAD, 16), _f32),
    mesh=_mesh,
    scratch_types=_AGG_SCRATCH,
    compiler_params=_SC_PARAMS,
)
def _agg2_kernel(src_hbm, dst_hbm, z_hbm, zeros_hbm, out_hbm,
                 sv, dv, rows, *sems_acc):
    gsem = list(sems_acc[:_NB])
    ssem = list(sems_acc[_NB:2 * _NB])
    acc = sems_acc[2 * _NB]
    cid = lax.axis_index("c")
    sid = lax.axis_index("s")
    pltpu.sync_copy(zeros_hbm.at[pl.ds(sid * DR, DR)],
                    acc.at[pl.ds(sid * DR, DR)])
    plsc.subcore_barrier()

    r0 = (cid * NS + sid) * NR2
    _edge_loop(z_hbm, src_hbm, dst_hbm, r0, NCH2, sv, dv, rows,
               gsem, ssem, acc)

    plsc.subcore_barrier()
    pltpu.sync_copy(acc.at[pl.ds(sid * DR, DR)],
                    out_hbm.at[pl.ds(cid * NPAD + sid * DR, DR)])


WN = NPAD * 16 // 128
SROWS = NPAD // 128
_RW = 1568
_GW = WN // _RW


def _mm_body(x_ref, wa_ref, wb_ref, oa_ref, ob_ref):
    x = x_ref[...]
    oa_ref[...] = jnp.dot(x, wa_ref[...], preferred_element_type=_f32)
    ob_ref[...] = jnp.dot(x, wb_ref[...], preferred_element_type=_f32)


def _s_body(deg_ref, s_ref):
    s_ref[...] = lax.rsqrt(deg_ref[0] + deg_ref[1] + 1.0)


def _zscale_body(xw0_ref, xw1_ref, s_ref, z0_ref, z1_ref):
    s = s_ref[...]
    z0_ref[...] = xw0_ref[...] * s
    z1_ref[...] = xw1_ref[...] * s


def _z2_body(a0_ref, a1_ref, z0_ref, z1_ref, s_ref, b1lo_ref, b1hi_ref,
             w2a_ref, w2b_ref, o_ref):
    s = s_ref[...]
    hlo = jnp.maximum(s * (a0_ref[...] + z0_ref[...]) + b1lo_ref[...], 0.0)
    hhi = jnp.maximum(s * (a1_ref[...] + z1_ref[...]) + b1hi_ref[...], 0.0)
    o_ref[...] = s * (jnp.dot(hlo, w2a_ref[...], preferred_element_type=_f32)
                      + jnp.dot(hhi, w2b_ref[...],
                                preferred_element_type=_f32))


def _out_body(a0_ref, a1_ref, z_ref, s_ref, b2_ref, wf_ref, bfc_ref, o_ref):
    h = jnp.maximum(
        s_ref[...] * (a0_ref[...] + a1_ref[...] + z_ref[...]) + b2_ref[...],
        0.0)
    o_ref[...] = (jnp.dot(h, wf_ref[...], preferred_element_type=_f32)
                  + bfc_ref[0, 0])


def _wspec(w=128):
    return pl.BlockSpec((_RW, w), lambda i: (i, 0))


def _whalf(half):
    return pl.BlockSpec((_RW, 128), lambda i, h=half: (h * _GW + i, 0))


def _full_spec(shape):
    return pl.BlockSpec(shape, lambda i: tuple(0 for _ in shape))


def kernel(edge_index, node_features, W1, b1, W2, b2, Wfc, bfc):
    src = edge_index[0]
    dst = edge_index[1]
    pad = N + (jnp.arange(EPAD - E, dtype=src.dtype) % 64)
    src_r = jnp.concatenate([src, pad]).reshape(RWS, 128)
    dst_r = jnp.concatenate([dst, pad]).reshape(RWS, 128)
    zeros1 = jnp.zeros((NPAD,), _f32)
    zeros2 = jnp.zeros((NPAD, 16), _f32)

    x_w = jnp.pad(node_features, ((0, NPAD - N), (0, 0))).reshape(WN, 128)
    e8 = jnp.eye(8, dtype=_f32)
    w1a = jnp.kron(e8, W1[:, :16])
    w1b = jnp.kron(e8, W1[:, 16:])
    w2a = jnp.kron(e8, W2[:16, :])
    w2b = jnp.kron(e8, W2[16:, :])
    wfk = jnp.kron(e8, Wfc)
    b1lo = jnp.tile(b1[:16], 8).reshape(1, 128)
    b1hi = jnp.tile(b1[16:], 8).reshape(1, 128)
    b2t = jnp.tile(b2, 8).reshape(1, 128)

    xw0, xw1 = pl.pallas_call(
        _mm_body,
        grid=(_GW,),
        in_specs=[_wspec(), _full_spec((128, 128)), _full_spec((128, 128))],
        out_specs=[_wspec(), _wspec()],
        out_shape=[jax.ShapeDtypeStruct((WN, 128), _f32)] * 2,
    )(x_w, w1a, w1b)

    deg2 = _deg_kernel(dst_r, zeros1).reshape(2, SROWS, 128)

    s_w = pl.pallas_call(
        _s_body,
        grid=(1,),
        in_specs=[_full_spec((2, SROWS, 128))],
        out_specs=_full_spec((SROWS, 128)),
        out_shape=jax.ShapeDtypeStruct((SROWS, 128), _f32),
    )(deg2)
    s16 = jnp.repeat(s_w.reshape(-1), 16).reshape(WN, 128)

    z0, z1 = pl.pallas_call(
        _zscale_body,
        grid=(_GW,),
        in_specs=[_wspec(), _wspec(), _wspec()],
        out_specs=[_wspec(), _wspec()],
        out_shape=[jax.ShapeDtypeStruct((WN, 128), _f32)] * 2,
    )(xw0, xw1, s16)

    agg1 = _agg1_kernel(src_r, dst_r, z0.reshape(NPAD, 16),
                        z1.reshape(NPAD, 16), zeros2).reshape(2 * WN, 128)

    z2 = pl.pallas_call(
        _z2_body,
        grid=(_GW,),
        in_specs=[_whalf(0), _whalf(1), _wspec(), _wspec(), _wspec(),
                  _full_spec((1, 128)), _full_spec((1, 128)),
                  _full_spec((128, 128)), _full_spec((128, 128))],
        out_specs=_wspec(),
        out_shape=jax.ShapeDtypeStruct((WN, 128), _f32),
    )(agg1, agg1, z0, z1, s16, b1lo, b1hi, w2a, w2b)

    agg2 = _agg2_kernel(src_r, dst_r, z2.reshape(NPAD, 16),
                        zeros2).reshape(2 * WN, 128)

    out = pl.pallas_call(
        _out_body,
        grid=(_GW,),
        in_specs=[_whalf(0), _whalf(1), _wspec(), _wspec(),
                  _full_spec((1, 128)), _full_spec((128, 8)),
                  _full_spec((1, 1))],
        out_specs=_wspec(8),
        out_shape=jax.ShapeDtypeStruct((WN, 8), _f32),
    )(agg2, agg2, z2, s16, b2t, wfk, bfc.reshape(1, 1))

    return out.reshape(NPAD, 1)[:N]


# speedup vs baseline: 2.2060x; 1.0509x over previous
import functools

import jax
import jax.numpy as jnp
from jax import lax
from jax.experimental import pallas as pl
from jax.experimental.pallas import tpu as pltpu
from jax.experimental.pallas import tpu_sc as plsc

N = 100000
NPAD = 100352
E = 3200000
EPAD = 3211264
RWS = EPAD // 128
NC, NS = 2, 16
NR1 = RWS // NS
NR2 = RWS // (NC * NS)
CH = 56
NCH1 = NR1 // CH
NCH2 = NR2 // CH
DR = NPAD // NS

_mesh = plsc.VectorSubcoreMesh(core_axis_name="c", subcore_axis_name="s")
_f32 = jnp.float32
_SC_PARAMS = pltpu.CompilerParams(use_tc_tiling_on_sc=False)

_NB = 7
_LEAD = 6


def _process_chunk(table, sv, dv, rows, gsem, ssem, acc):
    for j in range(_LEAD):
        pltpu.async_copy(table.at[sv.at[j]], rows.at[j], gsem[j])

    @pl.loop(0, CH // _NB)
    def _(q):
        k = q * _NB
        for j in range(_NB):
            i = k + j
            s2 = (j + _LEAD) % _NB
            pltpu.make_async_copy(table.at[sv.at[i]], rows.at[j],
                                  gsem[j]).wait()
            pltpu.async_copy(rows.at[j], acc.at[dv.at[i]], ssem[j],
                             add=True)

            @pl.when(i + _LEAD < CH)
            def _():
                @pl.when(i + _LEAD >= _NB)
                def _():
                    pltpu.make_async_copy(rows.at[s2], acc.at[dv.at[i]],
                                          ssem[s2]).wait()

                pltpu.async_copy(table.at[sv.at[i + _LEAD]], rows.at[s2],
                                 gsem[s2])

    for j in range(_NB):
        pltpu.make_async_copy(rows.at[j], acc.at[dv.at[0]], ssem[j]).wait()


def _edge_loop(table, src_hbm, dst_hbm, r0, nch, sv, dv, rows, gsem, ssem,
               acc):
    @pl.loop(0, nch)
    def _(c):
        base = r0 + c * CH
        pltpu.sync_copy(src_hbm.at[pl.ds(base, CH)], sv)
        pltpu.sync_copy(dst_hbm.at[pl.ds(base, CH)], dv)
        _process_chunk(table, sv, dv, rows, gsem, ssem, acc)


@functools.partial(
    pl.kernel,
    out_type=jax.ShapeDtypeStruct((NC * NPAD,), _f32),
    mesh=_mesh,
    scratch_types=[
        pltpu.VMEM((CH, 128), jnp.int32),
        pltpu.VMEM((128,), _f32),
        pltpu.SemaphoreType.DMA,
        pltpu.VMEM_SHARED((NPAD,), _f32),
    ],
    compiler_params=_SC_PARAMS,
)
def _deg_kernel(dst_hbm, zeros_hbm, out_hbm, dst_v, ones_v, dsem, acc):
    cid = lax.axis_index("c")
    sid = lax.axis_index("s")
    for i in range(8):
        ones_v[pl.ds(i * 16, 16)] = jnp.full((16,), 1.0, _f32)
    pltpu.sync_copy(zeros_hbm.at[pl.ds(sid * DR, DR)],
                    acc.at[pl.ds(sid * DR, DR)])
    plsc.subcore_barrier()

    r0 = (cid * NS + sid) * NR2

    @pl.loop(0, NCH2)
    def _(c):
        pltpu.sync_copy(dst_hbm.at[pl.ds(r0 + c * CH, CH)], dst_v)

        @pl.loop(0, 16)
        def _(k):
            pltpu.async_copy(ones_v, acc.at[dst_v.at[k]], dsem, add=True)

        @pl.loop(0, CH - 16)
        def _(k):
            pltpu.async_copy(ones_v, acc.at[dst_v.at[k + 16]], dsem,
                             add=True)
            pltpu.make_async_copy(ones_v, acc.at[dst_v.at[0]], dsem).wait()

        @pl.loop(0, 16)
        def _(k):
            pltpu.make_async_copy(ones_v, acc.at[dst_v.at[0]], dsem).wait()

    plsc.subcore_barrier()
    pltpu.sync_copy(acc.at[pl.ds(sid * DR, DR)],
                    out_hbm.at[pl.ds(cid * NPAD + sid * DR, DR)])


_AGG_SCRATCH = (
    [pltpu.VMEM((CH, 128), jnp.int32),
     pltpu.VMEM((CH, 128), jnp.int32),
     pltpu.VMEM((_NB, 128, 16), _f32)]
    + [pltpu.SemaphoreType.DMA] * (2 * _NB)
    + [pltpu.VMEM_SHARED((NPAD, 16), _f32)]
)


@functools.partial(
    pl.kernel,
    out_type=jax.ShapeDtypeStruct((NC * NPAD, 16), _f32),
    mesh=_mesh,
    scratch_types=_AGG_SCRATCH,
    compiler_params=_SC_PARAMS,
)
def _agg1_kernel(src_hbm, dst_hbm, z0_hbm, z1_hbm, zeros_hbm, out_hbm,
                 sv, dv, rows, *sems_acc):
    gsem = list(sems_acc[:_NB])
    ssem = list(sems_acc[_NB:2 * _NB])
    acc = sems_acc[2 * _NB]
    cid = lax.axis_index("c")
    sid = lax.axis_index("s")
    pltpu.sync_copy(zeros_hbm.at[pl.ds(sid * DR, DR)],
                    acc.at[pl.ds(sid * DR, DR)])
    plsc.subcore_barrier()

    r0 = sid * NR1

    @pl.when(cid == 0)
    def _():
        _edge_loop(z0_hbm, src_hbm, dst_hbm, r0, NCH1, sv, dv, rows,
                   gsem, ssem, acc)

    @pl.when(cid == 1)
    def _():
        _edge_loop(z1_hbm, src_hbm, dst_hbm, r0, NCH1, sv, dv, rows,
                   gsem, ssem, acc)

    plsc.subcore_barrier()
    pltpu.sync_copy(acc.at[pl.ds(sid * DR, DR)],
                    out_hbm.at[pl.ds(cid * NPAD + sid * DR, DR)])


@functools.partial(
    pl.kernel,
    out_type=jax.ShapeDtypeStruct((NC * NPAD, 16), _f32),
    mesh=_mesh,
    scratch_types=_AGG_SCRATCH,
    compiler_params=_SC_PARAMS,
)
def _agg2_kernel(src_hbm, dst_hbm, z_hbm, zeros_hbm, out_hbm,
                 sv, dv, rows, *sems_acc):
    gsem = list(sems_acc[:_NB])
    ssem = list(sems_acc[_NB:2 * _NB])
    acc = sems_acc[2 * _NB]
    cid = lax.axis_index("c")
    sid = lax.axis_index("s")
    pltpu.sync_copy(zeros_hbm.at[pl.ds(sid * DR, DR)],
                    acc.at[pl.ds(sid * DR, DR)])
    plsc.subcore_barrier()

    r0 = (cid * NS + sid) * NR2
    _edge_loop(z_hbm, src_hbm, dst_hbm, r0, NCH2, sv, dv, rows,
               gsem, ssem, acc)

    plsc.subcore_barrier()
    pltpu.sync_copy(acc.at[pl.ds(sid * DR, DR)],
                    out_hbm.at[pl.ds(cid * NPAD + sid * DR, DR)])


WN = NPAD * 16 // 128
SROWS = NPAD // 128
_RW = 1568
_GW = WN // _RW


def _mm_body(x_ref, wa_ref, wb_ref, oa_ref, ob_ref):
    x = x_ref[...]
    oa_ref[...] = jnp.dot(x, wa_ref[...], preferred_element_type=_f32)
    ob_ref[...] = jnp.dot(x, wb_ref[...], preferred_element_type=_f32)


def _s_body(deg_ref, s_ref):
    s_ref[...] = lax.rsqrt(deg_ref[0] + deg_ref[1] + 1.0)


def _zscale_body(xw0_ref, xw1_ref, s_ref, z0_ref, z1_ref):
    s = s_ref[...]
    z0_ref[...] = xw0_ref[...] * s
    z1_ref[...] = xw1_ref[...] * s


def _z2_body(a0_ref, a1_ref, z0_ref, z1_ref, s_ref, b1lo_ref, b1hi_ref,
             w2a_ref, w2b_ref, o_ref):
    s = s_ref[...]
    hlo = jnp.maximum(s * (a0_ref[...] + z0_ref[...]) + b1lo_ref[...], 0.0)
    hhi = jnp.maximum(s * (a1_ref[...] + z1_ref[...]) + b1hi_ref[...], 0.0)
    o_ref[...] = s * (jnp.dot(hlo, w2a_ref[...], preferred_element_type=_f32)
                      + jnp.dot(hhi, w2b_ref[...],
                                preferred_element_type=_f32))


def _out_body(a0_ref, a1_ref, z_ref, s_ref, b2_ref, wf_ref, bfc_ref, o_ref):
    h = jnp.maximum(
        s_ref[...] * (a0_ref[...] + a1_ref[...] + z_ref[...]) + b2_ref[...],
        0.0)
    o_ref[...] = (jnp.dot(h, wf_ref[...], preferred_element_type=_f32)
                  + bfc_ref[0, 0])


def _wspec(w=128):
    return pl.BlockSpec((_RW, w), lambda i: (i, 0))


def _whalf(half):
    return pl.BlockSpec((_RW, 128), lambda i, h=half: (h * _GW + i, 0))


def _full_spec(shape):
    return pl.BlockSpec(shape, lambda i: tuple(0 for _ in shape))


def kernel(edge_index, node_features, W1, b1, W2, b2, Wfc, bfc):
    src = edge_index[0]
    dst = edge_index[1]
    pad = N + (jnp.arange(EPAD - E, dtype=src.dtype) % 64)
    src_r = jnp.concatenate([src, pad]).reshape(RWS, 128)
    dst_r = jnp.concatenate([dst, pad]).reshape(RWS, 128)
    zeros1 = jnp.zeros((NPAD,), _f32)
    zeros2 = jnp.zeros((NPAD, 16), _f32)

    x_w = jnp.pad(node_features, ((0, NPAD - N), (0, 0))).reshape(WN, 128)
    e8 = jnp.eye(8, dtype=_f32)
    w1a = jnp.kron(e8, W1[:, :16])
    w1b = jnp.kron(e8, W1[:, 16:])
    w2a = jnp.kron(e8, W2[:16, :])
    w2b = jnp.kron(e8, W2[16:, :])
    wfk = jnp.kron(e8, Wfc)
    b1lo = jnp.tile(b1[:16], 8).reshape(1, 128)
    b1hi = jnp.tile(b1[16:], 8).reshape(1, 128)
    b2t = jnp.tile(b2, 8).reshape(1, 128)

    xw0, xw1 = pl.pallas_call(
        _mm_body,
        grid=(_GW,),
        in_specs=[_wspec(), _full_spec((128, 128)), _full_spec((128, 128))],
        out_specs=[_wspec(), _wspec()],
        out_shape=[jax.ShapeDtypeStruct((WN, 128), _f32)] * 2,
    )(x_w, w1a, w1b)

    deg2 = _deg_kernel(dst_r, zeros1).reshape(2, SROWS, 128)

    s_w = pl.pallas_call(
        _s_body,
        grid=(1,),
        in_specs=[_full_spec((2, SROWS, 128))],
        out_specs=_full_spec((SROWS, 128)),
        out_shape=jax.ShapeDtypeStruct((SROWS, 128), _f32),
    )(deg2)
    s16 = jnp.repeat(s_w.reshape(-1), 16).reshape(WN, 128)

    z0, z1 = pl.pallas_call(
        _zscale_body,
        grid=(_GW,),
        in_specs=[_wspec(), _wspec(), _wspec()],
        out_specs=[_wspec(), _wspec()],
        out_shape=[jax.ShapeDtypeStruct((WN, 128), _f32)] * 2,
    )(xw0, xw1, s16)

    agg1 = _agg1_kernel(src_r, dst_r, z0.reshape(NPAD, 16),
                        z1.reshape(NPAD, 16), zeros2).reshape(2 * WN, 128)

    z2 = pl.pallas_call(
        _z2_body,
        grid=(_GW,),
        in_specs=[_whalf(0), _whalf(1), _wspec(), _wspec(), _wspec(),
                  _full_spec((1, 128)), _full_spec((1, 128)),
                  _full_spec((128, 128)), _full_spec((128, 128))],
        out_specs=_wspec(),
        out_shape=jax.ShapeDtypeStruct((WN, 128), _f32),
    )(agg1, agg1, z0, z1, s16, b1lo, b1hi, w2a, w2b)

    agg2 = _agg2_kernel(src_r, dst_r, z2.reshape(NPAD, 16),
                        zeros2).reshape(2 * WN, 128)

    out = pl.pallas_call(
        _out_body,
        grid=(_GW,),
        in_specs=[_whalf(0), _whalf(1), _wspec(), _wspec(),
                  _full_spec((1, 128)), _full_spec((128, 8)),
                  _full_spec((1, 1))],
        out_specs=_wspec(8),
        out_shape=jax.ShapeDtypeStruct((WN, 8), _f32),
    )(agg2, agg2, z2, s16, b2t, wfk, bfc.reshape(1, 1))

    return out.reshape(NPAD, 1)[:N]
